# baseline (speedup 1.0000x reference)
import functools

import jax
import jax.numpy as jnp
from jax.experimental import pallas as pl
from jax.experimental.pallas import tpu as pltpu

NUM_EXPERTS = 8
SIZE = 224
PATCH = 16
DIM = 384
DEPTH = 6
HEADS = 8
DIM_HEAD = 64
MLP_DIM = 512
NUM_CLASSES = 10
BATCH = 32
NPATCH = (SIZE // PATCH) ** 2
PATCH_DIM = 3 * PATCH * PATCH
INNER = HEADS * DIM_HEAD
SEQ = NPATCH + 1

GATE_PREC = jax.lax.Precision.HIGHEST

GATE_CHUNKS = 8
GATE_K = 3 * SIZE * SIZE // GATE_CHUNKS


def _split_bf16(a):
    hi = a.astype(jnp.bfloat16)
    lo = (a - hi.astype(jnp.float32)).astype(jnp.bfloat16)
    return hi, lo


def _dot3(a, b, dims):
    ah, al = _split_bf16(a)
    bh, bl = _split_bf16(b)
    d = lambda u, v: jax.lax.dot_general(
        u, v, dims, preferred_element_type=jnp.float32)
    return d(ah, bh) + d(ah, bl) + d(al, bh)


def _mm(a, b):
    return _dot3(a, b, (((a.ndim - 1,), (0,)), ((), ())))


def _mm3p(a, bh, bl):
    ah, al = _split_bf16(a)
    dims = (((a.ndim - 1,), (0,)), ((), ()))
    d = lambda u, v: jax.lax.dot_general(
        u, v, dims, preferred_element_type=jnp.float32)
    return d(ah, bh) + d(ah, bl) + d(al, bh)


def _ln(x, g, b):
    m = jnp.mean(x, axis=-1, keepdims=True)
    v = jnp.mean((x - m) ** 2, axis=-1, keepdims=True)
    return (x - m) * jax.lax.rsqrt(v + 1e-5) * g + b


def _gate_kernel(xf_ref, gwt_ref, gb_ref, top1_ref, acc_ref):
    k = pl.program_id(0)

    @pl.when(k == 0)
    def _():
        acc_ref[...] = jnp.broadcast_to(gb_ref[...], (BATCH, NUM_EXPERTS))

    acc_ref[...] += jax.lax.dot_general(
        xf_ref[...], gwt_ref[...], (((1,), (1,)), ((), ())),
        precision=GATE_PREC, preferred_element_type=jnp.float32)

    @pl.when(k == GATE_CHUNKS - 1)
    def _():
        logits = acc_ref[...]
        m = jnp.max(logits, axis=1, keepdims=True)
        col = jax.lax.broadcasted_iota(jnp.int32, (BATCH, NUM_EXPERTS), 1)
        top1_ref[...] = jnp.min(
            jnp.where(logits == m, col, NUM_EXPERTS), axis=1, keepdims=True)


def _gate(xf, gwt, gb):
    return pl.pallas_call(
        _gate_kernel,
        grid=(GATE_CHUNKS,),
        in_specs=[
            pl.BlockSpec((BATCH, GATE_K), lambda k: (0, k)),
            pl.BlockSpec((NUM_EXPERTS, GATE_K), lambda k: (0, k)),
            pl.BlockSpec((1, NUM_EXPERTS), lambda k: (0, 0)),
        ],
        out_specs=pl.BlockSpec((BATCH, 1), lambda k: (0, 0)),
        out_shape=jax.ShapeDtypeStruct((BATCH, 1), jnp.int32),
        scratch_shapes=[pltpu.VMEM((BATCH, NUM_EXPERTS), jnp.float32)],
    )(xf, gwt, gb)


def _embed_kernel(se_ref, si_ref, xp_ref, g1_ref, b1_ref, pwh_ref, pwl_ref,
                  pb_ref, g2_ref, b2_ref, cls_ref, pos_ref, t0_ref):
    p = _ln(xp_ref[0], g1_ref[0, 0], b1_ref[0, 0])
    t = _mm3p(p, pwh_ref[0], pwl_ref[0]) + pb_ref[0, 0]
    t = _ln(t, g2_ref[0, 0], b2_ref[0, 0])
    t0_ref[0, 0:1] = cls_ref[0] + pos_ref[0, 0:1]
    t0_ref[0, 1:SEQ] = t + pos_ref[0, 1:SEQ]


def _embed(xp, g1, b1, pwh, pwl, pb, g2, b2, cls, pos, se, si):
    espec = lambda *blk: pl.BlockSpec((1,) + blk, lambda i, se, si: (se[i],) + (0,) * len(blk))
    return pl.pallas_call(
        _embed_kernel,
        grid_spec=pltpu.PrefetchScalarGridSpec(
            num_scalar_prefetch=2,
            grid=(BATCH,),
            in_specs=[
                pl.BlockSpec((1, NPATCH, PATCH_DIM), lambda i, se, si: (si[i], 0, 0)),
                espec(1, PATCH_DIM), espec(1, PATCH_DIM),
                espec(PATCH_DIM, DIM), espec(PATCH_DIM, DIM), espec(1, DIM),
                espec(1, DIM), espec(1, DIM),
                espec(1, DIM),
                espec(SEQ, DIM),
            ],
            out_specs=pl.BlockSpec((1, SEQ, DIM), lambda i, se, si: (i, 0, 0)),
        ),
        out_shape=jax.ShapeDtypeStruct((BATCH, SEQ, DIM), jnp.float32),
    )(se, si, xp, g1, b1, pwh, pwl, pb, g2, b2, cls, pos)


QCHUNK = SEQ


def _layer_body(b, mv_ref, qkv_ref, ow_ref, w1_ref, w2_ref,
                tbuf_ref, obuf_ref, j):
    mv = mv_ref[0, 0, 0]
    alg, alb = mv[0:DIM], mv[DIM:2 * DIM]
    ob = mv[2 * DIM:3 * DIM]
    flg, flb = mv[3 * DIM:4 * DIM], mv[4 * DIM:5 * DIM]
    b2 = mv[5 * DIM:6 * DIM]
    b1 = mv[6 * DIM:6 * DIM + MLP_DIM]

    t = tbuf_ref[b]
    y = _ln(t, alg, alb)
    qkv = _mm(y, qkv_ref[0, 0])
    scale = DIM_HEAD ** -0.5
    for h in range(HEADS):
        c = h * DIM_HEAD
        kh, kl = _split_bf16(qkv[:, INNER + c:INNER + c + DIM_HEAD])
        vh, vl = _split_bf16(qkv[:, 2 * INNER + c:2 * INNER + c + DIM_HEAD])
        qch, qcl = _split_bf16(qkv[:, c:c + DIM_HEAD])
        dk = lambda u, v: jax.lax.dot_general(
            u, v, (((1,), (1,)), ((), ())),
            preferred_element_type=jnp.float32)
        s = (dk(qch, kh) + dk(qch, kl) + dk(qcl, kh)) * scale
        s = jax.nn.softmax(s, axis=-1)
        sh, sl = _split_bf16(s)
        dv = lambda u, v: jax.lax.dot_general(
            u, v, (((1,), (0,)), ((), ())),
            preferred_element_type=jnp.float32)
        obuf_ref[j, :, c:c + DIM_HEAD] = dv(sh, vh) + dv(sh, vl) + dv(sl, vh)
    t = t + _mm(obuf_ref[j], ow_ref[0, 0]) + ob
    y = _ln(t, flg, flb)
    y = _mm(y, w1_ref[0, 0]) + b1
    y = 0.5 * y * (1.0 + jax.lax.erf(y * (2.0 ** -0.5)))
    t = t + _mm(y, w2_ref[0, 0]) + b2
    tbuf_ref[b] = t
    return t


def _layers_kernel(se_ref, si_ref, t0_ref,
                   mv0_ref, qkv0_ref, ow0_ref, w10_ref, w20_ref,
                   mv1_ref, qkv1_ref, ow1_ref, w11_ref, w21_ref,
                   fg_ref, fb_ref, hw_ref, hb_ref,
                   outa_ref, outb_ref, tbuf_ref, obuf_ref):
    l = pl.program_id(0)
    i = pl.program_id(1)

    @pl.when(l == 0)
    def _():
        tbuf_ref[pl.ds(2 * i, 2)] = t0_ref[...]

    ta = _layer_body(2 * i, mv0_ref, qkv0_ref, ow0_ref, w10_ref, w20_ref,
                     tbuf_ref, obuf_ref, 0)
    tb = _layer_body(2 * i + 1, mv1_ref, qkv1_ref, ow1_ref, w11_ref, w21_ref,
                     tbuf_ref, obuf_ref, 1)

    @pl.when(l == DEPTH - 1)
    def _():
        for t, out_ref, pos in ((ta, outa_ref, 2 * i), (tb, outb_ref, 2 * i + 1)):
            e = se_ref[pos]
            tf = _ln(t[0:1, :], fg_ref[e], fb_ref[e])
            out_ref[0] = _mm(tf, hw_ref[e]) + hb_ref[pl.ds(e, 1)]


def _layers(t0, mv, qkvw, ow, w1, w2, fg, fb, hw, hb, se, si):
    HB = BATCH // 2

    def lspec(j, *blk):
        return pl.BlockSpec(
            (1, 1) + blk,
            lambda l, i, se, si: (se[2 * i + j], l) + (0,) * len(blk))

    def vspec(j, d):
        return pl.BlockSpec(
            (1, 1, 1, d), lambda l, i, se, si: (se[2 * i + j], l, 0, 0))

    full = lambda arr: pl.BlockSpec(arr.shape, lambda l, i, se, si: (0,) * arr.ndim)

    def wgroup(j):
        return [
            vspec(j, 6 * DIM + MLP_DIM),
            lspec(j, DIM, 3 * INNER),
            lspec(j, INNER, DIM),
            lspec(j, DIM, MLP_DIM),
            lspec(j, MLP_DIM, DIM),
        ]

    def ospec(j):
        return pl.BlockSpec(
            (1, 1, NUM_CLASSES),
            lambda l, i, se, si: (
                jnp.where(l == DEPTH - 1, si[2 * i + j], si[j]), 0, 0))

    outs = pl.pallas_call(
        _layers_kernel,
        grid_spec=pltpu.PrefetchScalarGridSpec(
            num_scalar_prefetch=2,
            grid=(DEPTH, HB),
            in_specs=[
                pl.BlockSpec((2, SEQ, DIM),
                             lambda l, i, se, si: (jnp.where(l == 0, i, HB - 1), 0, 0)),
                *wgroup(0), *wgroup(1),
                full(fg), full(fb), full(hw), full(hb),
            ],
            out_specs=[ospec(0), ospec(1)],
            scratch_shapes=[pltpu.VMEM((BATCH, SEQ, DIM), jnp.float32),
                            pltpu.VMEM((2, SEQ, INNER), jnp.float32)],
        ),
        out_shape=[jax.ShapeDtypeStruct((BATCH, 1, NUM_CLASSES), jnp.float32)] * 2,
    )(se, si, t0, mv, qkvw, ow, w1, w2, mv, qkvw, ow, w1, w2,
      fg, fb, hw, hb)
    return outs


def kernel(x, params):
    experts = params['experts']
    stack = lambda key: jnp.stack([e[key] for e in experts])

    xf = x.reshape(BATCH, -1)
    h = SIZE // PATCH
    xp = x.reshape(BATCH, 3, h, PATCH, h, PATCH).transpose(
        0, 2, 4, 3, 5, 1).reshape(BATCH, NPATCH, PATCH_DIM)

    top1 = _gate(xf, params['gate_w'].T, params['gate_b'].reshape(1, -1))[:, 0]
    order = jnp.argsort(top1)
    se = top1[order].astype(jnp.int32)
    si = order.astype(jnp.int32)
    rank = jnp.argsort(si)

    vec = lambda key: stack(key)[:, None, :]
    lvec = lambda key: stack(key)[:, :, None, :]

    def hilo(key):
        w = stack(key)
        hi = w.astype(jnp.bfloat16)
        lo = (w - hi.astype(jnp.float32)).astype(jnp.bfloat16)
        return hi, lo

    pwh, pwl = hilo('pe_w')
    t0 = _embed(
        xp,
        vec('pe_ln1_g'), vec('pe_ln1_b'), pwh, pwl, vec('pe_b'),
        vec('pe_ln2_g'), vec('pe_ln2_b'),
        stack('cls').reshape(NUM_EXPERTS, 1, DIM),
        stack('pos').reshape(NUM_EXPERTS, SEQ, DIM),
        se, si)

    mv = jnp.concatenate(
        [stack('attn_ln_g'), stack('attn_ln_b'), stack('out_b'),
         stack('ff_ln_g'), stack('ff_ln_b'), stack('b2'), stack('b1')],
        axis=-1)[:, :, None, :]
    out_a, out_b = _layers(
        t0, mv, stack('qkv_w'), stack('out_w'), stack('w1'), stack('w2'),
        stack('final_ln_g'), stack('final_ln_b'),
        stack('head_w'), stack('head_b'),
        se, si)
    even = (rank % 2) == 0
    return jnp.where(even[:, None],
                     out_a.reshape(BATCH, NUM_CLASSES),
                     out_b.reshape(BATCH, NUM_CLASSES))


# speedup vs baseline: 1.0138x; 1.0138x over previous
import functools

import jax
import jax.numpy as jnp
from jax.experimental import pallas as pl
from jax.experimental.pallas import tpu as pltpu

NUM_EXPERTS = 8
SIZE = 224
PATCH = 16
DIM = 384
DEPTH = 6
HEADS = 8
DIM_HEAD = 64
MLP_DIM = 512
NUM_CLASSES = 10
BATCH = 32
NPATCH = (SIZE // PATCH) ** 2
PATCH_DIM = 3 * PATCH * PATCH
INNER = HEADS * DIM_HEAD
SEQ = NPATCH + 1

GATE_PREC = jax.lax.Precision.HIGHEST

GATE_CHUNKS = 8
GATE_K = 3 * SIZE * SIZE // GATE_CHUNKS


def _split_bf16(a):
    hi = a.astype(jnp.bfloat16)
    lo = (a - hi.astype(jnp.float32)).astype(jnp.bfloat16)
    return hi, lo


def _dot3(a, b, dims):
    ah, al = _split_bf16(a)
    bh, bl = _split_bf16(b)
    d = lambda u, v: jax.lax.dot_general(
        u, v, dims, preferred_element_type=jnp.float32)
    return d(ah, bh) + d(ah, bl) + d(al, bh)


def _mm(a, b):
    return _dot3(a, b, (((a.ndim - 1,), (0,)), ((), ())))


def _mm3p(a, bh, bl):
    ah, al = _split_bf16(a)
    dims = (((a.ndim - 1,), (0,)), ((), ()))
    d = lambda u, v: jax.lax.dot_general(
        u, v, dims, preferred_element_type=jnp.float32)
    return d(ah, bh) + d(ah, bl) + d(al, bh)


def _ln(x, g, b):
    m = jnp.mean(x, axis=-1, keepdims=True)
    v = jnp.mean((x - m) ** 2, axis=-1, keepdims=True)
    return (x - m) * jax.lax.rsqrt(v + 1e-5) * g + b


def _gate_kernel(xf_ref, gwt_ref, gb_ref, top1_ref, acc_ref):
    k = pl.program_id(0)

    @pl.when(k == 0)
    def _():
        acc_ref[...] = jnp.broadcast_to(gb_ref[...], (BATCH, NUM_EXPERTS))

    acc_ref[...] += jax.lax.dot_general(
        xf_ref[...], gwt_ref[...], (((1,), (1,)), ((), ())),
        precision=GATE_PREC, preferred_element_type=jnp.float32)

    @pl.when(k == GATE_CHUNKS - 1)
    def _():
        logits = acc_ref[...]
        m = jnp.max(logits, axis=1, keepdims=True)
        col = jax.lax.broadcasted_iota(jnp.int32, (BATCH, NUM_EXPERTS), 1)
        top1_ref[...] = jnp.min(
            jnp.where(logits == m, col, NUM_EXPERTS), axis=1, keepdims=True)


def _gate(xf, gwt, gb):
    return pl.pallas_call(
        _gate_kernel,
        grid=(GATE_CHUNKS,),
        in_specs=[
            pl.BlockSpec((BATCH, GATE_K), lambda k: (0, k)),
            pl.BlockSpec((NUM_EXPERTS, GATE_K), lambda k: (0, k)),
            pl.BlockSpec((1, NUM_EXPERTS), lambda k: (0, 0)),
        ],
        out_specs=pl.BlockSpec((BATCH, 1), lambda k: (0, 0)),
        out_shape=jax.ShapeDtypeStruct((BATCH, 1), jnp.int32),
        scratch_shapes=[pltpu.VMEM((BATCH, NUM_EXPERTS), jnp.float32)],
    )(xf, gwt, gb)


def _embed_kernel(se_ref, si_ref, xp_ref, g1_ref, b1_ref, pwh_ref, pwl_ref,
                  pb_ref, g2_ref, b2_ref, cls_ref, pos_ref, t0_ref):
    p = _ln(xp_ref[0], g1_ref[0, 0], b1_ref[0, 0])
    t = _mm3p(p, pwh_ref[0], pwl_ref[0]) + pb_ref[0, 0]
    t = _ln(t, g2_ref[0, 0], b2_ref[0, 0])
    t0_ref[0, 0:1] = cls_ref[0] + pos_ref[0, 0:1]
    t0_ref[0, 1:SEQ] = t + pos_ref[0, 1:SEQ]


def _embed(xp, g1, b1, pwh, pwl, pb, g2, b2, cls, pos, se, si):
    espec = lambda *blk: pl.BlockSpec((1,) + blk, lambda i, se, si: (se[i],) + (0,) * len(blk))
    return pl.pallas_call(
        _embed_kernel,
        grid_spec=pltpu.PrefetchScalarGridSpec(
            num_scalar_prefetch=2,
            grid=(BATCH,),
            in_specs=[
                pl.BlockSpec((1, NPATCH, PATCH_DIM), lambda i, se, si: (si[i], 0, 0)),
                espec(1, PATCH_DIM), espec(1, PATCH_DIM),
                espec(PATCH_DIM, DIM), espec(PATCH_DIM, DIM), espec(1, DIM),
                espec(1, DIM), espec(1, DIM),
                espec(1, DIM),
                espec(SEQ, DIM),
            ],
            out_specs=pl.BlockSpec((1, SEQ, DIM), lambda i, se, si: (i, 0, 0)),
        ),
        out_shape=jax.ShapeDtypeStruct((BATCH, SEQ, DIM), jnp.float32),
    )(se, si, xp, g1, b1, pwh, pwl, pb, g2, b2, cls, pos)


QCHUNK = SEQ


def _layer_body(b, mv_ref, qkv_ref, ow_ref, w1_ref, w2_ref,
                tbuf_ref, obuf_ref, j):
    mv = mv_ref[0, 0, 0]
    alg, alb = mv[0:DIM], mv[DIM:2 * DIM]
    ob = mv[2 * DIM:3 * DIM]
    flg, flb = mv[3 * DIM:4 * DIM], mv[4 * DIM:5 * DIM]
    b2 = mv[5 * DIM:6 * DIM]
    b1 = mv[6 * DIM:6 * DIM + MLP_DIM]

    t = tbuf_ref[b]
    y = _ln(t, alg, alb)
    qkv = _mm(y, qkv_ref[0, 0])
    scale = DIM_HEAD ** -0.5
    for h in range(HEADS):
        c = h * DIM_HEAD
        kh, kl = _split_bf16(qkv[:, INNER + c:INNER + c + DIM_HEAD])
        vh, vl = _split_bf16(qkv[:, 2 * INNER + c:2 * INNER + c + DIM_HEAD])
        qch, qcl = _split_bf16(qkv[:, c:c + DIM_HEAD])
        dk = lambda u, v: jax.lax.dot_general(
            u, v, (((1,), (1,)), ((), ())),
            preferred_element_type=jnp.float32)
        s = (dk(qch, kh) + dk(qch, kl) + dk(qcl, kh)) * scale
        s = jax.nn.softmax(s, axis=-1)
        sh, sl = _split_bf16(s)
        dv = lambda u, v: jax.lax.dot_general(
            u, v, (((1,), (0,)), ((), ())),
            preferred_element_type=jnp.float32)
        obuf_ref[j, :, c:c + DIM_HEAD] = dv(sh, vh) + dv(sh, vl) + dv(sl, vh)
    t = t + _mm(obuf_ref[j], ow_ref[0, 0]) + ob
    y = _ln(t, flg, flb)
    y = _mm(y, w1_ref[0, 0]) + b1
    y = 0.5 * y * (1.0 + jax.lax.erf(y * (2.0 ** -0.5)))
    t = t + _mm(y, w2_ref[0, 0]) + b2
    tbuf_ref[b] = t
    return t


NIMG = 4


def _layers_kernel(se_ref, si_ref, t0_ref, *rest):
    wrefs = rest[:5 * NIMG]
    fg_ref, fb_ref, hw_ref, hb_ref = rest[5 * NIMG:5 * NIMG + 4]
    out_refs = rest[5 * NIMG + 4:5 * NIMG + 4 + NIMG]
    tbuf_ref, obuf_ref = rest[5 * NIMG + 4 + NIMG:]
    l = pl.program_id(0)
    i = pl.program_id(1)

    @pl.when(l == 0)
    def _():
        tbuf_ref[pl.ds(NIMG * i, NIMG)] = t0_ref[...]

    ts = [
        _layer_body(NIMG * i + j, *wrefs[5 * j:5 * j + 5],
                    tbuf_ref, obuf_ref, j)
        for j in range(NIMG)
    ]

    @pl.when(l == DEPTH - 1)
    def _():
        for j in range(NIMG):
            e = se_ref[NIMG * i + j]
            tf = _ln(ts[j][0:1, :], fg_ref[e], fb_ref[e])
            out_refs[j][0] = _mm(tf, hw_ref[e]) + hb_ref[pl.ds(e, 1)]


def _layers(t0, mv, qkvw, ow, w1, w2, fg, fb, hw, hb, se, si):
    HB = BATCH // NIMG

    def lspec(j, *blk):
        return pl.BlockSpec(
            (1, 1) + blk,
            lambda l, i, se, si: (se[NIMG * i + j], l) + (0,) * len(blk))

    def vspec(j, d):
        return pl.BlockSpec(
            (1, 1, 1, d), lambda l, i, se, si: (se[NIMG * i + j], l, 0, 0))

    full = lambda arr: pl.BlockSpec(arr.shape, lambda l, i, se, si: (0,) * arr.ndim)

    def wgroup(j):
        return [
            vspec(j, 6 * DIM + MLP_DIM),
            lspec(j, DIM, 3 * INNER),
            lspec(j, INNER, DIM),
            lspec(j, DIM, MLP_DIM),
            lspec(j, MLP_DIM, DIM),
        ]

    def ospec(j):
        return pl.BlockSpec(
            (1, 1, NUM_CLASSES),
            lambda l, i, se, si: (
                jnp.where(l == DEPTH - 1, si[NIMG * i + j], si[j]), 0, 0))

    wspecs = []
    wargs = []
    for j in range(NIMG):
        wspecs.extend(wgroup(j))
        wargs.extend([mv, qkvw, ow, w1, w2])
    outs = pl.pallas_call(
        _layers_kernel,
        grid_spec=pltpu.PrefetchScalarGridSpec(
            num_scalar_prefetch=2,
            grid=(DEPTH, HB),
            in_specs=[
                pl.BlockSpec((NIMG, SEQ, DIM),
                             lambda l, i, se, si: (jnp.where(l == 0, i, HB - 1), 0, 0)),
                *wspecs,
                full(fg), full(fb), full(hw), full(hb),
            ],
            out_specs=[ospec(j) for j in range(NIMG)],
            scratch_shapes=[pltpu.VMEM((BATCH, SEQ, DIM), jnp.float32),
                            pltpu.VMEM((NIMG, SEQ, INNER), jnp.float32)],
        ),
        out_shape=[jax.ShapeDtypeStruct((BATCH, 1, NUM_CLASSES), jnp.float32)] * NIMG,
    )(se, si, t0, *wargs, fg, fb, hw, hb)
    return outs


def kernel(x, params):
    experts = params['experts']
    stack = lambda key: jnp.stack([e[key] for e in experts])

    xf = x.reshape(BATCH, -1)
    h = SIZE // PATCH
    xp = x.reshape(BATCH, 3, h, PATCH, h, PATCH).transpose(
        0, 2, 4, 3, 5, 1).reshape(BATCH, NPATCH, PATCH_DIM)

    top1 = _gate(xf, params['gate_w'].T, params['gate_b'].reshape(1, -1))[:, 0]
    order = jnp.argsort(top1)
    se = top1[order].astype(jnp.int32)
    si = order.astype(jnp.int32)
    rank = jnp.argsort(si)

    vec = lambda key: stack(key)[:, None, :]
    lvec = lambda key: stack(key)[:, :, None, :]

    def hilo(key):
        w = stack(key)
        hi = w.astype(jnp.bfloat16)
        lo = (w - hi.astype(jnp.float32)).astype(jnp.bfloat16)
        return hi, lo

    pwh, pwl = hilo('pe_w')
    t0 = _embed(
        xp,
        vec('pe_ln1_g'), vec('pe_ln1_b'), pwh, pwl, vec('pe_b'),
        vec('pe_ln2_g'), vec('pe_ln2_b'),
        stack('cls').reshape(NUM_EXPERTS, 1, DIM),
        stack('pos').reshape(NUM_EXPERTS, SEQ, DIM),
        se, si)

    mv = jnp.concatenate(
        [stack('attn_ln_g'), stack('attn_ln_b'), stack('out_b'),
         stack('ff_ln_g'), stack('ff_ln_b'), stack('b2'), stack('b1')],
        axis=-1)[:, :, None, :]
    outs = _layers(
        t0, mv, stack('qkv_w'), stack('out_w'), stack('w1'), stack('w2'),
        stack('final_ln_g'), stack('final_ln_b'),
        stack('head_w'), stack('head_b'),
        se, si)
    sel = rank % NIMG
    out = outs[0].reshape(BATCH, NUM_CLASSES)
    for j in range(1, NIMG):
        out = jnp.where((sel == j)[:, None],
                        outs[j].reshape(BATCH, NUM_CLASSES), out)
    return out
